# baseline (speedup 1.0000x reference)
import jax
import jax.numpy as jnp
from jax import lax
from jax.experimental import pallas as pl
from jax.experimental.pallas import tpu as pltpu

N_COLS = 32768
N_ROWS = 128
N_IDX = 16384

_N_LO = 128
_N_HI = N_COLS // _N_LO
_BR = 32


def _fused_body(idx_ref, x_ref, o_ref, mask_ref):
    i = pl.program_id(0)

    @pl.when(i == 0)
    def _():
        idx2 = idx_ref[...]
        hi = idx2 >> 7
        lo = idx2 & (_N_LO - 1)
        ah = (lax.broadcasted_iota(jnp.int32, (_N_HI, N_IDX), 0) == hi
              ).astype(jnp.bfloat16)
        bl = (lax.broadcasted_iota(jnp.int32, (_N_LO, N_IDX), 0) == lo
              ).astype(jnp.bfloat16)
        cnt = lax.dot_general(
            ah, bl, (((1,), (1,)), ((), ())),
            preferred_element_type=jnp.float32)
        mask_ref[...] = cnt.reshape(1, N_COLS)

    m = mask_ref[...] > 0.0
    o_ref[...] = jnp.where(m, x_ref[...], 0.0)


_fused = pl.pallas_call(
    _fused_body,
    grid=(N_ROWS // _BR,),
    in_specs=[
        pl.BlockSpec((1, N_IDX), lambda i: (0, 0)),
        pl.BlockSpec((_BR, N_COLS), lambda i: (i, 0)),
    ],
    out_specs=pl.BlockSpec((_BR, N_COLS), lambda i: (i, 0)),
    out_shape=jax.ShapeDtypeStruct((N_ROWS, N_COLS), jnp.float32),
    scratch_shapes=[pltpu.VMEM((1, N_COLS), jnp.float32)],
)


def kernel(x, retain_idx):
    idx2 = retain_idx.reshape(1, N_IDX)
    return _fused(idx2, x)


# speedup vs baseline: 1.1672x; 1.1672x over previous
import jax
import jax.numpy as jnp
from jax import lax
from jax.experimental import pallas as pl
from jax.experimental.pallas import tpu as pltpu

N_COLS = 32768
N_ROWS = 128
N_IDX = 16384

_N_LO = 128
_N_HI = N_COLS // _N_LO
_BR = 64


def _fused_body(idx_ref, x_ref, o_ref, mask_ref):
    i = pl.program_id(0)

    @pl.when(i == 0)
    def _():
        idx2 = idx_ref[...]
        hi = idx2 >> 7
        lo = idx2 & (_N_LO - 1)
        ah = (lax.broadcasted_iota(jnp.int32, (_N_HI, N_IDX), 0) == hi
              ).astype(jnp.bfloat16)
        bl = (lax.broadcasted_iota(jnp.int32, (_N_LO, N_IDX), 0) == lo
              ).astype(jnp.bfloat16)
        cnt = lax.dot_general(
            ah, bl, (((1,), (1,)), ((), ())),
            preferred_element_type=jnp.float32)
        mask_ref[...] = cnt.reshape(1, N_COLS)

    m = mask_ref[...] > 0.0
    o_ref[...] = jnp.where(m, x_ref[...], 0.0)


_fused = pl.pallas_call(
    _fused_body,
    grid=(N_ROWS // _BR,),
    in_specs=[
        pl.BlockSpec((1, N_IDX), lambda i: (0, 0)),
        pl.BlockSpec((_BR, N_COLS), lambda i: (i, 0)),
    ],
    out_specs=pl.BlockSpec((_BR, N_COLS), lambda i: (i, 0)),
    out_shape=jax.ShapeDtypeStruct((N_ROWS, N_COLS), jnp.float32),
    scratch_shapes=[pltpu.VMEM((1, N_COLS), jnp.float32)],
)


def kernel(x, retain_idx):
    idx2 = retain_idx.reshape(1, N_IDX)
    return _fused(idx2, x)


# speedup vs baseline: 1.2015x; 1.0294x over previous
import jax
import jax.numpy as jnp
from jax import lax
from jax.experimental import pallas as pl
from jax.experimental.pallas import tpu as pltpu

N_COLS = 32768
N_ROWS = 128
N_IDX = 16384

_N_LO = 128
_N_HI = N_COLS // _N_LO
_BN = 16384


def _fused_body(idx_ref, x_ref, o_ref, mask_ref):
    i = pl.program_id(0)

    @pl.when(i == 0)
    def _():
        idx2 = idx_ref[...]
        hi = idx2 >> 7
        lo = idx2 & (_N_LO - 1)
        ah = (lax.broadcasted_iota(jnp.int32, (_N_HI, N_IDX), 0) == hi
              ).astype(jnp.bfloat16)
        bl = (lax.broadcasted_iota(jnp.int32, (_N_LO, N_IDX), 0) == lo
              ).astype(jnp.bfloat16)
        cnt = lax.dot_general(
            ah, bl, (((1,), (1,)), ((), ())),
            preferred_element_type=jnp.float32)
        mask_ref[...] = cnt.reshape(1, N_COLS)

    m = mask_ref[0:1, pl.ds(i * _BN, _BN)] > 0.0
    o_ref[...] = jnp.where(m, x_ref[...], 0.0)


_fused = pl.pallas_call(
    _fused_body,
    grid=(N_COLS // _BN,),
    in_specs=[
        pl.BlockSpec((1, N_IDX), lambda i: (0, 0)),
        pl.BlockSpec((N_ROWS, _BN), lambda i: (0, i)),
    ],
    out_specs=pl.BlockSpec((N_ROWS, _BN), lambda i: (0, i)),
    out_shape=jax.ShapeDtypeStruct((N_ROWS, N_COLS), jnp.float32),
    scratch_shapes=[pltpu.VMEM((1, N_COLS), jnp.float32)],
)


def kernel(x, retain_idx):
    idx2 = retain_idx.reshape(1, N_IDX)
    return _fused(idx2, x)
